# Initial kernel scaffold; baseline (speedup 1.0000x reference)
import jax
import jax.numpy as jnp
from jax.experimental import pallas as pl


def kernel(x, W_map, b_map, W_router, b_router, W1, b1, g1, be1, W2, b2, g2, be2, W3, b3, g3, be3, W_out, b_out, g_out, be_out):
    raise NotImplementedError("write your pallas kernel here")


# speedup vs baseline: 2.1757x; 2.1757x over previous
import jax
import jax.numpy as jnp
from jax.experimental import pallas as pl
from jax.experimental.pallas import tpu as pltpu

B, T, D = 1, 2048, 1024
H = 2048
E = 8
OUT = 1024

_F32 = jnp.float32
_BF16 = jnp.bfloat16
_BIG = 3.0e38


_INV_SQRT2 = 0.7071067811865476


def _gelu_f32(v):
    return v * 0.5 * (1.0 + jax.lax.erf(v * _INV_SQRT2))


def _layernorm(h, g, b):
    mu = jnp.mean(h, axis=-1, keepdims=True)
    d = h - mu
    var = jnp.mean(d * d, axis=-1, keepdims=True)
    return d / jnp.sqrt(var + 1e-5) * g + b


def _router_kernel(x_ref, xb_ref, wm_ref, bm_ref, wr_ref, br_ref, s_ref):
    e = pl.program_id(0)
    t1 = jnp.dot(xb_ref[...], wm_ref[0], preferred_element_type=_F32)
    t1 = t1 + bm_ref[0]
    remb = jnp.dot(t1.astype(_BF16), wr_ref[...], preferred_element_type=_F32)
    remb = remb + br_ref[...]
    diff = x_ref[...] - remb
    d2 = jnp.sum(diff * diff, axis=1, keepdims=True)
    s = jnp.exp(-jnp.sqrt(d2))
    col = jax.lax.broadcasted_iota(jnp.int32, (T, E), 1)

    @pl.when(e == 0)
    def _():
        s_ref[...] = jnp.zeros_like(s_ref)

    s_ref[...] += jnp.where(col == e, s, 0.0)


def _router(x2d, xb, W_map, b_map, W_router, b_router):
    wm_b = W_map.astype(_BF16).reshape(D, E, D).swapaxes(0, 1)
    wr_b = W_router.astype(_BF16)
    bm = b_map.reshape(E, 1, D)
    br = b_router.reshape(1, D)
    return pl.pallas_call(
        _router_kernel,
        grid=(E,),
        in_specs=[
            pl.BlockSpec((T, D), lambda e: (0, 0)),
            pl.BlockSpec((T, D), lambda e: (0, 0)),
            pl.BlockSpec((1, D, D), lambda e: (e, 0, 0)),
            pl.BlockSpec((1, 1, D), lambda e: (e, 0, 0)),
            pl.BlockSpec((D, D), lambda e: (0, 0)),
            pl.BlockSpec((1, D), lambda e: (0, 0)),
        ],
        out_specs=pl.BlockSpec((T, E), lambda e: (0, 0)),
        out_shape=jax.ShapeDtypeStruct((T, E), _F32),
    )(x2d, xb, wm_b, bm, wr_b, br)


def _top2_kernel(s_ref, w_ref, ent_ref):
    s = s_ref[...]
    col = jax.lax.broadcasted_iota(jnp.int32, (T, E), 1)
    m1 = jnp.min(s, axis=1, keepdims=True)
    i1 = jnp.min(jnp.where(s == m1, col, E), axis=1, keepdims=True)
    s_m = jnp.where(col == i1, _BIG, s)
    m2 = jnp.min(s_m, axis=1, keepdims=True)
    i2 = jnp.min(jnp.where(s_m == m2, col, E), axis=1, keepdims=True)
    denom = m1 + m2
    p0 = m1 / denom
    p1 = m2 / denom
    w_ref[...] = jnp.where(col == i1, p0, 0.0) + jnp.where(col == i2, p1, 0.0)
    pc0 = jnp.clip(p0, 1e-6, 1.0)
    pc1 = jnp.clip(p1, 1e-6, 1.0)
    kd = dict(axis=(0, 1), keepdims=True)
    mar0 = jnp.sum(pc0, **kd) / (B * T)
    mar1 = jnp.sum(pc1, **kd) / (B * T)
    h_marg = -(mar0 * jnp.log(mar0) + mar1 * jnp.log(mar1))
    h_cond = jnp.sum(-(pc0 * jnp.log(pc0) + pc1 * jnp.log(pc1)), **kd) / (B * T)
    ent_ref[...] = -(h_marg - h_cond)


def _top2(scores):
    return pl.pallas_call(
        _top2_kernel,
        out_shape=(
            jax.ShapeDtypeStruct((T, E), _F32),
            jax.ShapeDtypeStruct((1, 1), _F32),
        ),
    )(scores)


_TB = 512
_NTB = T // _TB


def _ffn_layer_kernel(hin_ref, w_ref, b_ref, g_ref, be_ref, out_ref):
    h = jnp.dot(hin_ref[0], w_ref[0], preferred_element_type=_F32)
    h = h + b_ref[0]
    h = _gelu_f32(h)
    h = _layernorm(h, g_ref[0], be_ref[0])
    out_ref[0] = h.astype(_BF16)


def _ffn_layer(hin, W, b, g, be, din, dout, shared_in=False):
    in_map = (lambda e, t: (0, t, 0)) if shared_in else (lambda e, t: (e, t, 0))
    return pl.pallas_call(
        _ffn_layer_kernel,
        grid=(E, _NTB),
        in_specs=[
            pl.BlockSpec((1, _TB, din), in_map),
            pl.BlockSpec((1, din, dout), lambda e, t: (e, 0, 0)),
            pl.BlockSpec((1, 1, dout), lambda e, t: (e, 0, 0)),
            pl.BlockSpec((1, 1, dout), lambda e, t: (e, 0, 0)),
            pl.BlockSpec((1, 1, dout), lambda e, t: (e, 0, 0)),
        ],
        out_specs=pl.BlockSpec((1, _TB, dout), lambda e, t: (e, t, 0)),
        out_shape=jax.ShapeDtypeStruct((E, T, dout), _BF16),
    )(hin, W, b.reshape(E, 1, dout), g.reshape(E, 1, dout), be.reshape(E, 1, dout))


def _combine_kernel(h3_ref, wgt_ref, wout_ref, bout_ref, gout_ref, beout_ref,
                    out_ref):
    acc = jnp.zeros((_TB, H), _F32)
    for e in range(E):
        we = wgt_ref[:, e:e + 1]
        acc = acc + we * h3_ref[e].astype(_F32)
    o = jnp.dot(acc.astype(_BF16), wout_ref[...], preferred_element_type=_F32)
    o = o + bout_ref[...]
    o = _gelu_f32(o)
    out_ref[...] = _layernorm(o, gout_ref[...], beout_ref[...])


def _combine(h3, wgt, W_out, b_out, g_out, be_out):
    return pl.pallas_call(
        _combine_kernel,
        grid=(_NTB,),
        in_specs=[
            pl.BlockSpec((E, _TB, H), lambda t: (0, t, 0)),
            pl.BlockSpec((_TB, E), lambda t: (t, 0)),
            pl.BlockSpec((H, OUT), lambda t: (0, 0)),
            pl.BlockSpec((1, OUT), lambda t: (0, 0)),
            pl.BlockSpec((1, OUT), lambda t: (0, 0)),
            pl.BlockSpec((1, OUT), lambda t: (0, 0)),
        ],
        out_specs=pl.BlockSpec((_TB, OUT), lambda t: (t, 0)),
        out_shape=jax.ShapeDtypeStruct((T, OUT), _F32),
    )(h3, wgt, W_out.astype(_BF16), b_out.reshape(1, OUT),
      g_out.reshape(1, OUT), be_out.reshape(1, OUT))


def kernel(x, W_map, b_map, W_router, b_router, W1, b1, g1, be1, W2, b2, g2,
           be2, W3, b3, g3, be3, W_out, b_out, g_out, be_out):
    x2d = x.reshape(T, D)
    xb = x2d.astype(_BF16)

    scores = _router(x2d, xb, W_map, b_map, W_router, b_router)
    wgt, ent = _top2(scores)

    h1 = _ffn_layer(xb[None], W1.astype(_BF16), b1, g1, be1, D, H,
                    shared_in=True)
    h2 = _ffn_layer(h1, W2.astype(_BF16), b2, g2, be2, H, H)
    h3 = _ffn_layer(h2, W3.astype(_BF16), b3, g3, be3, H, H)

    outs = _combine(h3, wgt, W_out, b_out, g_out, be_out)
    return outs.reshape(B, T, OUT), ent[0, 0]


# speedup vs baseline: 3.0932x; 1.4217x over previous
import jax
import jax.numpy as jnp
from jax.experimental import pallas as pl
from jax.experimental.pallas import tpu as pltpu

B, T, D = 1, 2048, 1024
H = 2048
E = 8
OUT = 1024

_RB = 256
_NB = 24
_RT = _NB * _RB
_TB = 256

_F32 = jnp.float32
_BF16 = jnp.bfloat16
_I32 = jnp.int32
_BIG = 3.0e38
_INV_SQRT2 = 0.7071067811865476


def _gelu_f32(v):
    return v * 0.5 * (1.0 + jax.lax.erf(v * _INV_SQRT2))


def _layernorm(h, g, b):
    mu = jnp.mean(h, axis=-1, keepdims=True)
    d = h - mu
    var = jnp.mean(d * d, axis=-1, keepdims=True)
    return d / jnp.sqrt(var + 1e-5) * g + b


def _router_kernel(x_ref, xb_ref, wm_ref, bm_ref, wr_ref, br_ref, s_ref):
    e = pl.program_id(0)
    t1 = jnp.dot(xb_ref[...], wm_ref[0], preferred_element_type=_F32)
    t1 = t1 + bm_ref[0]
    remb = jnp.dot(t1.astype(_BF16), wr_ref[...], preferred_element_type=_F32)
    remb = remb + br_ref[...]
    diff = x_ref[...] - remb
    d2 = jnp.sum(diff * diff, axis=1, keepdims=True)
    s = jnp.exp(-jnp.sqrt(d2))
    col = jax.lax.broadcasted_iota(_I32, (T, E), 1)

    @pl.when(e == 0)
    def _():
        s_ref[...] = jnp.zeros_like(s_ref)

    s_ref[...] += jnp.where(col == e, s, 0.0)


def _router(x2d, xb, W_map, b_map, W_router, b_router):
    wm_b = W_map.astype(_BF16).reshape(D, E, D).swapaxes(0, 1)
    wr_b = W_router.astype(_BF16)
    bm = b_map.reshape(E, 1, D)
    br = b_router.reshape(1, D)
    return pl.pallas_call(
        _router_kernel,
        grid=(E,),
        in_specs=[
            pl.BlockSpec((T, D), lambda e: (0, 0)),
            pl.BlockSpec((T, D), lambda e: (0, 0)),
            pl.BlockSpec((1, D, D), lambda e: (e, 0, 0)),
            pl.BlockSpec((1, 1, D), lambda e: (e, 0, 0)),
            pl.BlockSpec((D, D), lambda e: (0, 0)),
            pl.BlockSpec((1, D), lambda e: (0, 0)),
        ],
        out_specs=pl.BlockSpec((T, E), lambda e: (0, 0)),
        out_shape=jax.ShapeDtypeStruct((T, E), _F32),
    )(x2d, xb, wm_b, bm, wr_b, br)


def _routing_kernel(s_ref, tri_ref, posr_ref, pp_ref, emap_ref, nblk_ref,
                    ent_ref):
    s = s_ref[...]
    col = jax.lax.broadcasted_iota(_I32, (T, E), 1)
    m1 = jnp.min(s, axis=1, keepdims=True)
    i1 = jnp.min(jnp.where(s == m1, col, E), axis=1, keepdims=True)
    s_m = jnp.where(col == i1, _BIG, s)
    m2 = jnp.min(s_m, axis=1, keepdims=True)
    i2 = jnp.min(jnp.where(s_m == m2, col, E), axis=1, keepdims=True)
    denom = m1 + m2
    p0 = m1 / denom
    p1 = m2 / denom

    pc0 = jnp.clip(p0, 1e-6, 1.0)
    pc1 = jnp.clip(p1, 1e-6, 1.0)
    kd = dict(axis=(0, 1), keepdims=True)
    mar0 = jnp.sum(pc0, **kd) / (B * T)
    mar1 = jnp.sum(pc1, **kd) / (B * T)
    h_marg = -(mar0 * jnp.log(mar0) + mar1 * jnp.log(mar1))
    h_cond = jnp.sum(-(pc0 * jnp.log(pc0) + pc1 * jnp.log(pc1)), **kd) / (B * T)
    ent_ref[...] = -(h_marg - h_cond)

    oh0 = (col == i1)
    oh1 = (col == i2)
    oh01 = jnp.where(oh0 | oh1, 1.0, 0.0).astype(_BF16)
    cnt_excl = jnp.dot(tri_ref[...], oh01, preferred_element_type=_F32)
    counts = jnp.sum(oh01.astype(_F32), axis=0, keepdims=True)
    nb = jnp.floor((counts + (_RB - 1)) * (1.0 / _RB))
    e_r = jax.lax.broadcasted_iota(_I32, (E, E), 0)
    e_c = jax.lax.broadcasted_iota(_I32, (E, E), 1)
    nb8 = jnp.broadcast_to(nb, (E, E))
    nb_s = jnp.sum(jnp.where(e_r == e_c, nb8, 0.0), axis=1, keepdims=True)
    nb_sb = jnp.broadcast_to(nb_s, (E, E))
    boff = jnp.sum(jnp.where(e_r < e_c, nb_sb, 0.0), axis=0, keepdims=True)
    boff = boff * float(_RB)
    rank0 = jnp.sum(jnp.where(oh0, cnt_excl + boff, 0.0), axis=1,
                    keepdims=True)
    rank1 = jnp.sum(jnp.where(oh1, cnt_excl + boff, 0.0), axis=1,
                    keepdims=True)
    pp_ref[...] = jnp.concatenate([rank0, rank1, p0, p1], axis=1)
    posr = jnp.concatenate([rank0, rank1], axis=1)
    posr_ref[...] = jnp.transpose(posr, (1, 0))

    cb_incl = jnp.sum(jnp.where(e_r <= e_c, jnp.broadcast_to(nb_s, (E, E)),
                                0.0), axis=0, keepdims=True)
    b_iota = jax.lax.broadcasted_iota(_I32, (E, 128), 1).astype(_F32)
    cb_s = jnp.sum(jnp.where(e_r == e_c, jnp.broadcast_to(cb_incl, (E, E)),
                             0.0), axis=1, keepdims=True)
    emap = jnp.sum(jnp.where(b_iota >= jnp.broadcast_to(cb_s, (E, 128)),
                             1.0, 0.0), axis=0, keepdims=True)
    emap_ref[...] = jnp.minimum(emap, float(E - 1)).astype(_I32)
    nblk_ref[...] = jnp.sum(nb_s, axis=(0, 1), keepdims=True).astype(_I32)


def _routing(scores, tri):
    return pl.pallas_call(
        _routing_kernel,
        out_shape=(
            jax.ShapeDtypeStruct((2, T), _F32),
            jax.ShapeDtypeStruct((T, 4), _F32),
            jax.ShapeDtypeStruct((1, 128), _I32),
            jax.ShapeDtypeStruct((1, 1), _I32),
            jax.ShapeDtypeStruct((1, 1), _F32),
        ),
    )(scores, tri)


def _l1_kernel(emap_ref, nblk_ref, xb_ref, posr_ref, w_ref, b_ref, g_ref,
               be_ref, o_ref):
    b = pl.program_id(0)

    @pl.when(b < nblk_ref[0])
    def _():
        riota = jax.lax.broadcasted_iota(_I32, (_RB, T), 0) + b * _RB
        rf = riota.astype(_F32)
        hit = (posr_ref[0:1, :] == rf) | (posr_ref[1:2, :] == rf)
        oh = jnp.where(hit, 1.0, 0.0).astype(_BF16)
        xg = jnp.dot(oh, xb_ref[...], preferred_element_type=_F32)
        h = jnp.dot(xg.astype(_BF16), w_ref[0], preferred_element_type=_F32)
        h = _gelu_f32(h + b_ref[0])
        h = _layernorm(h, g_ref[0], be_ref[0])
        o_ref[...] = h.astype(_BF16)
    @pl.when(b >= nblk_ref[0])
    def _():
        o_ref[...] = jnp.zeros_like(o_ref)


def _l1(emap, nblk, xb, posr, W1, b1, g1, be1):
    return pl.pallas_call(
        _l1_kernel,
        grid_spec=pltpu.PrefetchScalarGridSpec(
            num_scalar_prefetch=2,
            grid=(_NB,),
            in_specs=[
                pl.BlockSpec((T, D), lambda b, em, nb_: (0, 0)),
                pl.BlockSpec((2, T), lambda b, em, nb_: (0, 0)),
                pl.BlockSpec((1, D, H), lambda b, em, nb_: (em[b], 0, 0)),
                pl.BlockSpec((1, 1, H), lambda b, em, nb_: (em[b], 0, 0)),
                pl.BlockSpec((1, 1, H), lambda b, em, nb_: (em[b], 0, 0)),
                pl.BlockSpec((1, 1, H), lambda b, em, nb_: (em[b], 0, 0)),
            ],
            out_specs=pl.BlockSpec((_RB, H), lambda b, em, nb_: (b, 0)),
        ),
        out_shape=jax.ShapeDtypeStruct((_RT, H), _BF16),
    )(emap, nblk, xb, posr, W1, b1.reshape(E, 1, H), g1.reshape(E, 1, H),
      be1.reshape(E, 1, H))


def _mid_kernel(emap_ref, nblk_ref, hin_ref, w_ref, b_ref, g_ref, be_ref,
                o_ref):
    b = pl.program_id(0)

    @pl.when(b < nblk_ref[0])
    def _():
        h = jnp.dot(hin_ref[...], w_ref[0], preferred_element_type=_F32)
        h = _gelu_f32(h + b_ref[0])
        h = _layernorm(h, g_ref[0], be_ref[0])
        o_ref[...] = h.astype(_BF16)
    @pl.when(b >= nblk_ref[0])
    def _():
        o_ref[...] = jnp.zeros_like(o_ref)


def _mid(emap, nblk, hin, W, bb, g, be):
    return pl.pallas_call(
        _mid_kernel,
        grid_spec=pltpu.PrefetchScalarGridSpec(
            num_scalar_prefetch=2,
            grid=(_NB,),
            in_specs=[
                pl.BlockSpec((_RB, H), lambda b, em, nb_: (b, 0)),
                pl.BlockSpec((1, H, H), lambda b, em, nb_: (em[b], 0, 0)),
                pl.BlockSpec((1, 1, H), lambda b, em, nb_: (em[b], 0, 0)),
                pl.BlockSpec((1, 1, H), lambda b, em, nb_: (em[b], 0, 0)),
                pl.BlockSpec((1, 1, H), lambda b, em, nb_: (em[b], 0, 0)),
            ],
            out_specs=pl.BlockSpec((_RB, H), lambda b, em, nb_: (b, 0)),
        ),
        out_shape=jax.ShapeDtypeStruct((_RT, H), _BF16),
    )(emap, nblk, hin, W, bb.reshape(E, 1, H), g.reshape(E, 1, H),
      be.reshape(E, 1, H))


def _l3_kernel(emap_ref, nblk_ref, hin_ref, w_ref, b_ref, g_ref, be_ref,
               wout_ref, o_ref):
    b = pl.program_id(0)

    @pl.when(b < nblk_ref[0])
    def _():
        h = jnp.dot(hin_ref[...], w_ref[0], preferred_element_type=_F32)
        h = _gelu_f32(h + b_ref[0])
        h = _layernorm(h, g_ref[0], be_ref[0])
        y = jnp.dot(h.astype(_BF16), wout_ref[...],
                    preferred_element_type=_F32)
        o_ref[...] = y.astype(_BF16)
    @pl.when(b >= nblk_ref[0])
    def _():
        o_ref[...] = jnp.zeros_like(o_ref)


def _l3(emap, nblk, hin, W, bb, g, be, W_out):
    return pl.pallas_call(
        _l3_kernel,
        grid_spec=pltpu.PrefetchScalarGridSpec(
            num_scalar_prefetch=2,
            grid=(_NB,),
            in_specs=[
                pl.BlockSpec((_RB, H), lambda b, em, nb_: (b, 0)),
                pl.BlockSpec((1, H, H), lambda b, em, nb_: (em[b], 0, 0)),
                pl.BlockSpec((1, 1, H), lambda b, em, nb_: (em[b], 0, 0)),
                pl.BlockSpec((1, 1, H), lambda b, em, nb_: (em[b], 0, 0)),
                pl.BlockSpec((1, 1, H), lambda b, em, nb_: (em[b], 0, 0)),
                pl.BlockSpec((H, OUT), lambda b, em, nb_: (0, 0)),
            ],
            out_specs=pl.BlockSpec((_RB, OUT), lambda b, em, nb_: (b, 0)),
        ),
        out_shape=jax.ShapeDtypeStruct((_RT, OUT), _BF16),
    )(emap, nblk, hin, W, bb.reshape(E, 1, H), g.reshape(E, 1, H),
      be.reshape(E, 1, H), W_out.astype(_BF16))


def _final_kernel(y_ref, pp_ref, bout_ref, gout_ref, beout_ref, o_ref):
    pos0 = pp_ref[:, 0:1]
    pos1 = pp_ref[:, 1:2]
    p0 = pp_ref[:, 2:3]
    p1 = pp_ref[:, 3:4]
    liota = jax.lax.broadcasted_iota(_I32, (_TB, _RT), 1).astype(_F32)
    gmat = jnp.where(liota == pos0, p0, 0.0) + jnp.where(liota == pos1, p1,
                                                         0.0)
    comb = jnp.dot(gmat.astype(_BF16), y_ref[...],
                   preferred_element_type=_F32)
    o = _gelu_f32(comb + bout_ref[...])
    o_ref[...] = _layernorm(o, gout_ref[...], beout_ref[...])


def _final(y, pp, b_out, g_out, be_out):
    return pl.pallas_call(
        _final_kernel,
        grid=(T // _TB,),
        in_specs=[
            pl.BlockSpec((_RT, OUT), lambda t: (0, 0)),
            pl.BlockSpec((_TB, 4), lambda t: (t, 0)),
            pl.BlockSpec((1, OUT), lambda t: (0, 0)),
            pl.BlockSpec((1, OUT), lambda t: (0, 0)),
            pl.BlockSpec((1, OUT), lambda t: (0, 0)),
        ],
        out_specs=pl.BlockSpec((_TB, OUT), lambda t: (t, 0)),
        out_shape=jax.ShapeDtypeStruct((T, OUT), _F32),
    )(y, pp, b_out.reshape(1, OUT), g_out.reshape(1, OUT),
      be_out.reshape(1, OUT))


def kernel(x, W_map, b_map, W_router, b_router, W1, b1, g1, be1, W2, b2, g2,
           be2, W3, b3, g3, be3, W_out, b_out, g_out, be_out):
    x2d = x.reshape(T, D)
    xb = x2d.astype(_BF16)

    scores = _router(x2d, xb, W_map, b_map, W_router, b_router)

    r = jax.lax.broadcasted_iota(_I32, (T, T), 0)
    c = jax.lax.broadcasted_iota(_I32, (T, T), 1)
    tri = jnp.where(c < r, 1.0, 0.0).astype(_BF16)

    posr, pp, emap_w, nblk11, ent = _routing(scores, tri)
    emap = emap_w.reshape(128)
    nblk = nblk11.reshape(1)

    h1 = _l1(emap, nblk, xb, posr, W1.astype(_BF16), b1, g1, be1)
    h2 = _mid(emap, nblk, h1, W2.astype(_BF16), b2, g2, be2)
    y = _l3(emap, nblk, h2, W3.astype(_BF16), b3, g3, be3, W_out)
    outs = _final(y, pp, b_out, g_out, be_out)
    return outs.reshape(B, T, OUT), ent[0, 0]


# speedup vs baseline: 3.1859x; 1.0299x over previous
import jax
import jax.numpy as jnp
from jax.experimental import pallas as pl
from jax.experimental.pallas import tpu as pltpu
from jax.experimental.pallas import tpu_sc as plsc
import functools

B, T, D = 1, 2048, 1024
H = 2048
E = 8
OUT = 1024

_RB = 256
_NB = 24
_RT = _NB * _RB
_TB = 256

_F32 = jnp.float32
_BF16 = jnp.bfloat16
_I32 = jnp.int32
_BIG = 3.0e38
_INV_SQRT2 = 0.7071067811865476


def _gelu_f32(v):
    return v * 0.5 * (1.0 + jax.lax.erf(v * _INV_SQRT2))


def _layernorm(h, g, b):
    mu = jnp.mean(h, axis=-1, keepdims=True)
    d = h - mu
    var = jnp.mean(d * d, axis=-1, keepdims=True)
    return d / jnp.sqrt(var + 1e-5) * g + b


def _rmap_kernel(xb_ref, wm_ref, bm_ref, o_ref):
    t1 = jnp.dot(xb_ref[...], wm_ref[...], preferred_element_type=_F32)
    o_ref[...] = (t1 + bm_ref[0]).astype(_BF16)


def _rdist_kernel(x_ref, t1_ref, wr_ref, br_ref, s_ref):
    e = pl.program_id(0)
    remb = jnp.dot(t1_ref[...], wr_ref[...], preferred_element_type=_F32)
    remb = remb + br_ref[...]
    diff = x_ref[...] - remb
    d2 = jnp.sum(diff * diff, axis=1, keepdims=True)
    s = jnp.exp(-jnp.sqrt(d2))
    col = jax.lax.broadcasted_iota(_I32, (T, E), 1)

    @pl.when(e == 0)
    def _():
        s_ref[...] = jnp.zeros_like(s_ref)

    s_ref[...] += jnp.where(col == e, s, 0.0)


def _router(x2d, xb, W_map, b_map, W_router, b_router):
    wm_b = W_map.astype(_BF16)
    wr_b = W_router.astype(_BF16)
    br = b_router.reshape(1, D)
    t1 = pl.pallas_call(
        _rmap_kernel,
        grid=(E,),
        in_specs=[
            pl.BlockSpec((T, D), lambda e: (0, 0)),
            pl.BlockSpec((D, D), lambda e: (0, e)),
            pl.BlockSpec((1, 1, D), lambda e: (e, 0, 0)),
        ],
        out_specs=pl.BlockSpec((T, D), lambda e: (0, e)),
        out_shape=jax.ShapeDtypeStruct((T, E * D), _BF16),
    )(xb, wm_b, b_map.reshape(E, 1, D))
    return pl.pallas_call(
        _rdist_kernel,
        grid=(E,),
        in_specs=[
            pl.BlockSpec((T, D), lambda e: (0, 0)),
            pl.BlockSpec((T, D), lambda e: (0, e)),
            pl.BlockSpec((D, D), lambda e: (0, 0)),
            pl.BlockSpec((1, D), lambda e: (0, 0)),
        ],
        out_specs=pl.BlockSpec((T, E), lambda e: (0, 0)),
        out_shape=jax.ShapeDtypeStruct((T, E), _F32),
    )(x2d, t1, wr_b, br)


def _routing_kernel(s_ref, tri_ref, posr_ref, pp_ref, emap_ref, nblk_ref,
                    ent_ref):
    s = s_ref[...]
    col = jax.lax.broadcasted_iota(_I32, (T, E), 1)
    m1 = jnp.min(s, axis=1, keepdims=True)
    i1 = jnp.min(jnp.where(s == m1, col, E), axis=1, keepdims=True)
    s_m = jnp.where(col == i1, _BIG, s)
    m2 = jnp.min(s_m, axis=1, keepdims=True)
    i2 = jnp.min(jnp.where(s_m == m2, col, E), axis=1, keepdims=True)
    denom = m1 + m2
    p0 = m1 / denom
    p1 = m2 / denom

    pc0 = jnp.clip(p0, 1e-6, 1.0)
    pc1 = jnp.clip(p1, 1e-6, 1.0)
    kd = dict(axis=(0, 1), keepdims=True)
    mar0 = jnp.sum(pc0, **kd) / (B * T)
    mar1 = jnp.sum(pc1, **kd) / (B * T)
    h_marg = -(mar0 * jnp.log(mar0) + mar1 * jnp.log(mar1))
    h_cond = jnp.sum(-(pc0 * jnp.log(pc0) + pc1 * jnp.log(pc1)), **kd) / (B * T)
    ent_ref[...] = -(h_marg - h_cond)

    oh0 = (col == i1)
    oh1 = (col == i2)
    oh01 = jnp.where(oh0 | oh1, 1.0, 0.0).astype(_BF16)
    cnt_excl = jnp.dot(tri_ref[...], oh01, preferred_element_type=_F32)
    counts = jnp.sum(oh01.astype(_F32), axis=0, keepdims=True)
    nb = jnp.floor((counts + (_RB - 1)) * (1.0 / _RB))
    e_r = jax.lax.broadcasted_iota(_I32, (E, E), 0)
    e_c = jax.lax.broadcasted_iota(_I32, (E, E), 1)
    nb8 = jnp.broadcast_to(nb, (E, E))
    nb_s = jnp.sum(jnp.where(e_r == e_c, nb8, 0.0), axis=1, keepdims=True)
    nb_sb = jnp.broadcast_to(nb_s, (E, E))
    boff = jnp.sum(jnp.where(e_r < e_c, nb_sb, 0.0), axis=0, keepdims=True)
    boff = boff * float(_RB)
    rank0 = jnp.sum(jnp.where(oh0, cnt_excl + boff, 0.0), axis=1,
                    keepdims=True)
    rank1 = jnp.sum(jnp.where(oh1, cnt_excl + boff, 0.0), axis=1,
                    keepdims=True)
    pp_ref[...] = jnp.concatenate([rank0, rank1, p0, p1], axis=1)
    posr = jnp.concatenate([rank0, rank1], axis=1)
    posr_ref[...] = jnp.transpose(posr, (1, 0))

    cb_incl = jnp.sum(jnp.where(e_r <= e_c, jnp.broadcast_to(nb_s, (E, E)),
                                0.0), axis=0, keepdims=True)
    b_iota = jax.lax.broadcasted_iota(_I32, (E, 128), 1).astype(_F32)
    cb_s = jnp.sum(jnp.where(e_r == e_c, jnp.broadcast_to(cb_incl, (E, E)),
                             0.0), axis=1, keepdims=True)
    emap = jnp.sum(jnp.where(b_iota >= jnp.broadcast_to(cb_s, (E, 128)),
                             1.0, 0.0), axis=0, keepdims=True)
    emap_ref[...] = jnp.minimum(emap, float(E - 1)).astype(_I32)
    nblk_ref[...] = jnp.sum(nb_s, axis=(0, 1), keepdims=True).astype(_I32)


def _routing(scores, tri):
    return pl.pallas_call(
        _routing_kernel,
        out_shape=(
            jax.ShapeDtypeStruct((2, T), _F32),
            jax.ShapeDtypeStruct((T, 4), _F32),
            jax.ShapeDtypeStruct((1, 128), _I32),
            jax.ShapeDtypeStruct((1, 1), _I32),
            jax.ShapeDtypeStruct((1, 1), _F32),
        ),
    )(scores, tri)


def _l1_kernel(emap_ref, nblk_ref, xb_ref, posr_ref, w_ref, b_ref, g_ref,
               be_ref, o_ref):
    b = pl.program_id(0)

    @pl.when(b < nblk_ref[0])
    def _():
        riota = jax.lax.broadcasted_iota(_I32, (_RB, T), 0) + b * _RB
        rf = riota.astype(_F32)
        hit = (posr_ref[0:1, :] == rf) | (posr_ref[1:2, :] == rf)
        oh = jnp.where(hit, 1.0, 0.0).astype(_BF16)
        xg = jnp.dot(oh, xb_ref[...], preferred_element_type=_F32)
        h = jnp.dot(xg.astype(_BF16), w_ref[0], preferred_element_type=_F32)
        h = _gelu_f32(h + b_ref[0])
        h = _layernorm(h, g_ref[0], be_ref[0])
        o_ref[...] = h.astype(_BF16)
    @pl.when(b >= nblk_ref[0])
    def _():
        o_ref[...] = jnp.zeros_like(o_ref)


def _l1(emap, nblk, xb, posr, W1, b1, g1, be1):
    return pl.pallas_call(
        _l1_kernel,
        grid_spec=pltpu.PrefetchScalarGridSpec(
            num_scalar_prefetch=2,
            grid=(_NB,),
            in_specs=[
                pl.BlockSpec((T, D), lambda b, em, nb_: (0, 0)),
                pl.BlockSpec((2, T), lambda b, em, nb_: (0, 0)),
                pl.BlockSpec((1, D, H), lambda b, em, nb_: (em[b], 0, 0)),
                pl.BlockSpec((1, 1, H), lambda b, em, nb_: (em[b], 0, 0)),
                pl.BlockSpec((1, 1, H), lambda b, em, nb_: (em[b], 0, 0)),
                pl.BlockSpec((1, 1, H), lambda b, em, nb_: (em[b], 0, 0)),
            ],
            out_specs=pl.BlockSpec((_RB, H), lambda b, em, nb_: (b, 0)),
        ),
        out_shape=jax.ShapeDtypeStruct((_RT, H), _BF16),
    )(emap, nblk, xb, posr, W1, b1.reshape(E, 1, H), g1.reshape(E, 1, H),
      be1.reshape(E, 1, H))


def _mid_kernel(emap_ref, nblk_ref, hin_ref, w_ref, b_ref, g_ref, be_ref,
                o_ref):
    b = pl.program_id(0)

    @pl.when(b < nblk_ref[0])
    def _():
        h = jnp.dot(hin_ref[...], w_ref[0], preferred_element_type=_F32)
        h = _gelu_f32(h + b_ref[0])
        h = _layernorm(h, g_ref[0], be_ref[0])
        o_ref[...] = h.astype(_BF16)
    @pl.when(b >= nblk_ref[0])
    def _():
        o_ref[...] = jnp.zeros_like(o_ref)


def _mid(emap, nblk, hin, W, bb, g, be):
    return pl.pallas_call(
        _mid_kernel,
        grid_spec=pltpu.PrefetchScalarGridSpec(
            num_scalar_prefetch=2,
            grid=(_NB,),
            in_specs=[
                pl.BlockSpec((_RB, H), lambda b, em, nb_: (b, 0)),
                pl.BlockSpec((1, H, H), lambda b, em, nb_: (em[b], 0, 0)),
                pl.BlockSpec((1, 1, H), lambda b, em, nb_: (em[b], 0, 0)),
                pl.BlockSpec((1, 1, H), lambda b, em, nb_: (em[b], 0, 0)),
                pl.BlockSpec((1, 1, H), lambda b, em, nb_: (em[b], 0, 0)),
            ],
            out_specs=pl.BlockSpec((_RB, H), lambda b, em, nb_: (b, 0)),
        ),
        out_shape=jax.ShapeDtypeStruct((_RT, H), _BF16),
    )(emap, nblk, hin, W, bb.reshape(E, 1, H), g.reshape(E, 1, H),
      be.reshape(E, 1, H))


def _l3_kernel(emap_ref, nblk_ref, hin_ref, w_ref, b_ref, g_ref, be_ref,
               wout_ref, o0_ref, o1_ref, o2_ref, o3_ref):
    b = pl.program_id(0)
    qs = OUT // 4

    @pl.when(b < nblk_ref[0])
    def _():
        h = jnp.dot(hin_ref[...], w_ref[0], preferred_element_type=_F32)
        h = _gelu_f32(h + b_ref[0])
        h = _layernorm(h, g_ref[0], be_ref[0])
        y = jnp.dot(h.astype(_BF16), wout_ref[...],
                    preferred_element_type=_F32)
        o0_ref[...] = y[:, 0 * qs:1 * qs]
        o1_ref[...] = y[:, 1 * qs:2 * qs]
        o2_ref[...] = y[:, 2 * qs:3 * qs]
        o3_ref[...] = y[:, 3 * qs:4 * qs]

    @pl.when(b >= nblk_ref[0])
    def _():
        o0_ref[...] = jnp.zeros_like(o0_ref)
        o1_ref[...] = jnp.zeros_like(o1_ref)
        o2_ref[...] = jnp.zeros_like(o2_ref)
        o3_ref[...] = jnp.zeros_like(o3_ref)


def _l3(emap, nblk, hin, W, bb, g, be, W_out):
    qs = OUT // 4
    return pl.pallas_call(
        _l3_kernel,
        grid_spec=pltpu.PrefetchScalarGridSpec(
            num_scalar_prefetch=2,
            grid=(_NB,),
            in_specs=[
                pl.BlockSpec((_RB, H), lambda b, em, nb_: (b, 0)),
                pl.BlockSpec((1, H, H), lambda b, em, nb_: (em[b], 0, 0)),
                pl.BlockSpec((1, 1, H), lambda b, em, nb_: (em[b], 0, 0)),
                pl.BlockSpec((1, 1, H), lambda b, em, nb_: (em[b], 0, 0)),
                pl.BlockSpec((1, 1, H), lambda b, em, nb_: (em[b], 0, 0)),
                pl.BlockSpec((H, OUT), lambda b, em, nb_: (0, 0)),
            ],
            out_specs=[
                pl.BlockSpec((_RB, qs), lambda b, em, nb_: (b, 0)),
                pl.BlockSpec((_RB, qs), lambda b, em, nb_: (b, 0)),
                pl.BlockSpec((_RB, qs), lambda b, em, nb_: (b, 0)),
                pl.BlockSpec((_RB, qs), lambda b, em, nb_: (b, 0)),
            ],
        ),
        out_shape=[jax.ShapeDtypeStruct((_RT, qs), _F32) for _ in range(4)],
    )(emap, nblk, hin, W, bb.reshape(E, 1, H), g.reshape(E, 1, H),
      be.reshape(E, 1, H), W_out.astype(_BF16))


_SC_WIN = 128


def _sc_gather(yq, idx):
    qs = OUT // 4
    mesh = plsc.VectorSubcoreMesh(core_axis_name="c", subcore_axis_name="s")

    @functools.partial(
        pl.kernel,
        out_type=[jax.ShapeDtypeStruct((2 * T, qs), _F32) for _ in range(4)],
        mesh=mesh,
    )
    def run(y0, y1, y2, y3, i_hbm, o0, o1, o2, o3):
        for y_hbm, o_hbm in ((y0, o0), (y1, o1), (y2, o2), (y3, o3)):
            def body(i_vmem, o_vmem, y_hbm=y_hbm):
                pltpu.sync_copy(y_hbm.at[i_vmem.at[0]], o_vmem)

            pltpu.emit_pipeline(
                body,
                grid=(2 * T // _SC_WIN,),
                in_specs=[pl.BlockSpec((1, _SC_WIN), lambda i: (0, i))],
                out_specs=[pl.BlockSpec((_SC_WIN, qs), lambda i: (i, 0))],
                core_axis_name=("c", "s"),
                dimension_semantics=(pltpu.PARALLEL,),
            )(i_hbm, o_hbm)

    return run(*yq, idx)


def _final_kernel(y00, y01, y02, y03, y10, y11, y12, y13, pp_ref, bout_ref,
                  gout_ref, beout_ref, o_ref):
    p0 = pp_ref[:, 2:3]
    p1 = pp_ref[:, 3:4]
    y0 = jnp.concatenate([y00[...], y01[...], y02[...], y03[...]], axis=1)
    y1 = jnp.concatenate([y10[...], y11[...], y12[...], y13[...]], axis=1)
    comb = p0 * y0 + p1 * y1
    o = _gelu_f32(comb + bout_ref[...])
    o_ref[...] = _layernorm(o, gout_ref[...], beout_ref[...])


def _final(ygq, pp, b_out, g_out, be_out):
    nt = T // _TB
    qs = OUT // 4
    qspec0 = pl.BlockSpec((_TB, qs), lambda t: (t, 0))
    qspec1 = pl.BlockSpec((_TB, qs), lambda t, nt=nt: (t + nt, 0))
    return pl.pallas_call(
        _final_kernel,
        grid=(nt,),
        in_specs=[qspec0] * 4 + [qspec1] * 4 + [
            pl.BlockSpec((_TB, 4), lambda t: (t, 0)),
            pl.BlockSpec((1, OUT), lambda t: (0, 0)),
            pl.BlockSpec((1, OUT), lambda t: (0, 0)),
            pl.BlockSpec((1, OUT), lambda t: (0, 0)),
        ],
        out_specs=pl.BlockSpec((_TB, OUT), lambda t: (t, 0)),
        out_shape=jax.ShapeDtypeStruct((T, OUT), _F32),
    )(*ygq, *ygq, pp, b_out.reshape(1, OUT), g_out.reshape(1, OUT),
      be_out.reshape(1, OUT))


def kernel(x, W_map, b_map, W_router, b_router, W1, b1, g1, be1, W2, b2, g2,
           be2, W3, b3, g3, be3, W_out, b_out, g_out, be_out):
    x2d = x.reshape(T, D)
    xb = x2d.astype(_BF16)

    scores = _router(x2d, xb, W_map, b_map, W_router, b_router)

    r = jax.lax.broadcasted_iota(_I32, (T, T), 0)
    c = jax.lax.broadcasted_iota(_I32, (T, T), 1)
    tri = jnp.where(c < r, 1.0, 0.0).astype(_BF16)

    posr, pp, emap_w, nblk11, ent = _routing(scores, tri)
    emap = emap_w.reshape(128)
    nblk = nblk11.reshape(1)

    h1 = _l1(emap, nblk, xb, posr, W1.astype(_BF16), b1, g1, be1)
    h2 = _mid(emap, nblk, h1, W2.astype(_BF16), b2, g2, be2)
    yq = _l3(emap, nblk, h2, W3.astype(_BF16), b3, g3, be3, W_out)
    idx = posr.reshape(1, 2 * T).astype(_I32)
    ygq = _sc_gather(yq, idx)
    outs = _final(ygq, pp, b_out, g_out, be_out)
    return outs.reshape(B, T, OUT), ent[0, 0]


# speedup vs baseline: 3.4233x; 1.0745x over previous
import jax
import jax.numpy as jnp
from jax.experimental import pallas as pl
from jax.experimental.pallas import tpu as pltpu
from jax.experimental.pallas import tpu_sc as plsc
import functools

B, T, D = 1, 2048, 1024
H = 2048
E = 8
OUT = 1024

_RB = 256
_NB = 24
_RT = _NB * _RB
_TB = 256

_F32 = jnp.float32
_BF16 = jnp.bfloat16
_I32 = jnp.int32
_BIG = 3.0e38
_INV_SQRT2 = 0.7071067811865476


def _gelu_f32(v):
    return v * 0.5 * (1.0 + jax.lax.erf(v * _INV_SQRT2))


def _layernorm(h, g, b):
    mu = jnp.mean(h, axis=-1, keepdims=True)
    d = h - mu
    var = jnp.mean(d * d, axis=-1, keepdims=True)
    return d / jnp.sqrt(var + 1e-5) * g + b


def _rmap_kernel(xb_ref, wm_ref, bm_ref, o_ref):
    t1 = jnp.dot(xb_ref[...], wm_ref[...], preferred_element_type=_F32)
    o_ref[...] = (t1 + bm_ref[0]).astype(_BF16)


def _rdist_kernel(x_ref, t1_ref, wr_ref, br_ref, s_ref):
    e = pl.program_id(0)
    remb = jnp.dot(t1_ref[...], wr_ref[...], preferred_element_type=_F32)
    remb = remb + br_ref[...]
    diff = x_ref[...] - remb
    d2 = jnp.sum(diff * diff, axis=1, keepdims=True)
    s = jnp.exp(-jnp.sqrt(d2))
    col = jax.lax.broadcasted_iota(_I32, (T, E), 1)

    @pl.when(e == 0)
    def _():
        s_ref[...] = jnp.zeros_like(s_ref)

    s_ref[...] += jnp.where(col == e, s, 0.0)


def _router(x2d, xb, W_map, b_map, W_router, b_router):
    wm_b = W_map.astype(_BF16)
    wr_b = W_router.astype(_BF16)
    br = b_router.reshape(1, D)
    t1 = pl.pallas_call(
        _rmap_kernel,
        grid=(E,),
        in_specs=[
            pl.BlockSpec((T, D), lambda e: (0, 0)),
            pl.BlockSpec((D, D), lambda e: (0, e)),
            pl.BlockSpec((1, 1, D), lambda e: (e, 0, 0)),
        ],
        out_specs=pl.BlockSpec((T, D), lambda e: (0, e)),
        out_shape=jax.ShapeDtypeStruct((T, E * D), _BF16),
    )(xb, wm_b, b_map.reshape(E, 1, D))
    return pl.pallas_call(
        _rdist_kernel,
        grid=(E,),
        in_specs=[
            pl.BlockSpec((T, D), lambda e: (0, 0)),
            pl.BlockSpec((T, D), lambda e: (0, e)),
            pl.BlockSpec((D, D), lambda e: (0, 0)),
            pl.BlockSpec((1, D), lambda e: (0, 0)),
        ],
        out_specs=pl.BlockSpec((T, E), lambda e: (0, 0)),
        out_shape=jax.ShapeDtypeStruct((T, E), _F32),
    )(x2d, t1, wr_b, br)


def _routing_kernel(s_ref, tri_ref, posr_ref, pp_ref, emap_ref, nblk_ref,
                    ent_ref):
    s = s_ref[...]
    col = jax.lax.broadcasted_iota(_I32, (T, E), 1)
    m1 = jnp.min(s, axis=1, keepdims=True)
    i1 = jnp.min(jnp.where(s == m1, col, E), axis=1, keepdims=True)
    s_m = jnp.where(col == i1, _BIG, s)
    m2 = jnp.min(s_m, axis=1, keepdims=True)
    i2 = jnp.min(jnp.where(s_m == m2, col, E), axis=1, keepdims=True)
    denom = m1 + m2
    p0 = m1 / denom
    p1 = m2 / denom

    pc0 = jnp.clip(p0, 1e-6, 1.0)
    pc1 = jnp.clip(p1, 1e-6, 1.0)
    kd = dict(axis=(0, 1), keepdims=True)
    mar0 = jnp.sum(pc0, **kd) / (B * T)
    mar1 = jnp.sum(pc1, **kd) / (B * T)
    h_marg = -(mar0 * jnp.log(mar0) + mar1 * jnp.log(mar1))
    h_cond = jnp.sum(-(pc0 * jnp.log(pc0) + pc1 * jnp.log(pc1)), **kd) / (B * T)
    ent_ref[...] = -(h_marg - h_cond)

    oh0 = (col == i1)
    oh1 = (col == i2)
    oh01 = jnp.where(oh0 | oh1, 1.0, 0.0).astype(_BF16)
    cnt_excl = jnp.dot(tri_ref[...], oh01, preferred_element_type=_F32)
    counts = jnp.sum(oh01.astype(_F32), axis=0, keepdims=True)
    nb = jnp.floor((counts + (_RB - 1)) * (1.0 / _RB))
    e_r = jax.lax.broadcasted_iota(_I32, (E, E), 0)
    e_c = jax.lax.broadcasted_iota(_I32, (E, E), 1)
    nb8 = jnp.broadcast_to(nb, (E, E))
    nb_s = jnp.sum(jnp.where(e_r == e_c, nb8, 0.0), axis=1, keepdims=True)
    nb_sb = jnp.broadcast_to(nb_s, (E, E))
    boff = jnp.sum(jnp.where(e_r < e_c, nb_sb, 0.0), axis=0, keepdims=True)
    boff = boff * float(_RB)
    rank0 = jnp.sum(jnp.where(oh0, cnt_excl + boff, 0.0), axis=1,
                    keepdims=True)
    rank1 = jnp.sum(jnp.where(oh1, cnt_excl + boff, 0.0), axis=1,
                    keepdims=True)
    pp_ref[...] = jnp.concatenate([rank0, rank1, p0, p1], axis=1)
    posr = jnp.concatenate([rank0, rank1], axis=1)
    posr_ref[...] = jnp.transpose(posr, (1, 0))

    cb_incl = jnp.sum(jnp.where(e_r <= e_c, jnp.broadcast_to(nb_s, (E, E)),
                                0.0), axis=0, keepdims=True)
    b_iota = jax.lax.broadcasted_iota(_I32, (E, 128), 1).astype(_F32)
    cb_s = jnp.sum(jnp.where(e_r == e_c, jnp.broadcast_to(cb_incl, (E, E)),
                             0.0), axis=1, keepdims=True)
    emap = jnp.sum(jnp.where(b_iota >= jnp.broadcast_to(cb_s, (E, 128)),
                             1.0, 0.0), axis=0, keepdims=True)
    emap_ref[...] = jnp.minimum(emap, float(E - 1)).astype(_I32)
    nblk_ref[...] = jnp.sum(nb_s, axis=(0, 1), keepdims=True).astype(_I32)


def _routing(scores, tri):
    return pl.pallas_call(
        _routing_kernel,
        out_shape=(
            jax.ShapeDtypeStruct((2, T), _F32),
            jax.ShapeDtypeStruct((T, 4), _F32),
            jax.ShapeDtypeStruct((1, 128), _I32),
            jax.ShapeDtypeStruct((1, 1), _I32),
            jax.ShapeDtypeStruct((1, 1), _F32),
        ),
    )(scores, tri)


def _ffn_kernel(emap_ref, nblk_ref, xb_ref, posr_ref,
                w1_ref, b1_ref, g1_ref, be1_ref,
                w2_ref, b2_ref, g2_ref, be2_ref,
                w3_ref, b3_ref, g3_ref, be3_ref,
                wout_ref, o0_ref, o1_ref, o2_ref, o3_ref):
    b = pl.program_id(0)
    qs = OUT // 4

    @pl.when(b < nblk_ref[0])
    def _():
        riota = jax.lax.broadcasted_iota(_I32, (_RB, T), 0) + b * _RB
        rf = riota.astype(_F32)
        hit = (posr_ref[0:1, :] == rf) | (posr_ref[1:2, :] == rf)
        oh = jnp.where(hit, 1.0, 0.0).astype(_BF16)
        xg = jnp.dot(oh, xb_ref[...], preferred_element_type=_F32)
        h = jnp.dot(xg.astype(_BF16), w1_ref[0], preferred_element_type=_F32)
        h = _layernorm(_gelu_f32(h + b1_ref[0]), g1_ref[0], be1_ref[0])
        h = jnp.dot(h.astype(_BF16), w2_ref[0], preferred_element_type=_F32)
        h = _layernorm(_gelu_f32(h + b2_ref[0]), g2_ref[0], be2_ref[0])
        h = jnp.dot(h.astype(_BF16), w3_ref[0], preferred_element_type=_F32)
        h = _layernorm(_gelu_f32(h + b3_ref[0]), g3_ref[0], be3_ref[0])
        y = jnp.dot(h.astype(_BF16), wout_ref[...],
                    preferred_element_type=_F32)
        o0_ref[...] = y[:, 0 * qs:1 * qs]
        o1_ref[...] = y[:, 1 * qs:2 * qs]
        o2_ref[...] = y[:, 2 * qs:3 * qs]
        o3_ref[...] = y[:, 3 * qs:4 * qs]

    @pl.when(b >= nblk_ref[0])
    def _():
        o0_ref[...] = jnp.zeros_like(o0_ref)
        o1_ref[...] = jnp.zeros_like(o1_ref)
        o2_ref[...] = jnp.zeros_like(o2_ref)
        o3_ref[...] = jnp.zeros_like(o3_ref)


def _ffn(emap, nblk, xb, posr, W1, b1, g1, be1, W2, b2, g2, be2, W3, b3, g3,
         be3, W_out):
    qs = OUT // 4
    em_spec = lambda shape: pl.BlockSpec(shape, lambda b, em, nb_: (em[b], 0, 0))
    vspec = em_spec((1, 1, H))
    return pl.pallas_call(
        _ffn_kernel,
        grid_spec=pltpu.PrefetchScalarGridSpec(
            num_scalar_prefetch=2,
            grid=(_NB,),
            in_specs=[
                pl.BlockSpec((T, D), lambda b, em, nb_: (0, 0)),
                pl.BlockSpec((2, T), lambda b, em, nb_: (0, 0)),
                em_spec((1, D, H)), vspec, vspec, vspec,
                em_spec((1, H, H)), vspec, vspec, vspec,
                em_spec((1, H, H)), vspec, vspec, vspec,
                pl.BlockSpec((H, OUT), lambda b, em, nb_: (0, 0)),
            ],
            out_specs=[
                pl.BlockSpec((_RB, qs), lambda b, em, nb_: (b, 0))
                for _ in range(4)
            ],
        ),
        out_shape=[jax.ShapeDtypeStruct((_RT, qs), _F32) for _ in range(4)],
    )(emap, nblk, xb, posr,
      W1.astype(_BF16), b1.reshape(E, 1, H), g1.reshape(E, 1, H),
      be1.reshape(E, 1, H),
      W2.astype(_BF16), b2.reshape(E, 1, H), g2.reshape(E, 1, H),
      be2.reshape(E, 1, H),
      W3.astype(_BF16), b3.reshape(E, 1, H), g3.reshape(E, 1, H),
      be3.reshape(E, 1, H),
      W_out.astype(_BF16))


_SC_WIN = 128


def _sc_gather(yq, idx):
    qs = OUT // 4
    mesh = plsc.VectorSubcoreMesh(core_axis_name="c", subcore_axis_name="s")

    @functools.partial(
        pl.kernel,
        out_type=[jax.ShapeDtypeStruct((2 * T, qs), _F32) for _ in range(4)],
        mesh=mesh,
    )
    def run(y0, y1, y2, y3, i_hbm, o0, o1, o2, o3):
        for y_hbm, o_hbm in ((y0, o0), (y1, o1), (y2, o2), (y3, o3)):
            def body(i_vmem, o_vmem, y_hbm=y_hbm):
                pltpu.sync_copy(y_hbm.at[i_vmem.at[0]], o_vmem)

            pltpu.emit_pipeline(
                body,
                grid=(2 * T // _SC_WIN,),
                in_specs=[pl.BlockSpec((1, _SC_WIN), lambda i: (0, i))],
                out_specs=[pl.BlockSpec((_SC_WIN, qs), lambda i: (i, 0))],
                core_axis_name=("c", "s"),
                dimension_semantics=(pltpu.PARALLEL,),
            )(i_hbm, o_hbm)

    return run(*yq, idx)


def _final_kernel(y00, y01, y02, y03, y10, y11, y12, y13, pp_ref, bout_ref,
                  gout_ref, beout_ref, o_ref):
    p0 = pp_ref[:, 2:3]
    p1 = pp_ref[:, 3:4]
    y0 = jnp.concatenate([y00[...], y01[...], y02[...], y03[...]], axis=1)
    y1 = jnp.concatenate([y10[...], y11[...], y12[...], y13[...]], axis=1)
    comb = p0 * y0 + p1 * y1
    o = _gelu_f32(comb + bout_ref[...])
    o_ref[...] = _layernorm(o, gout_ref[...], beout_ref[...])


def _final(ygq, pp, b_out, g_out, be_out):
    nt = T // _TB
    qs = OUT // 4
    qspec0 = pl.BlockSpec((_TB, qs), lambda t: (t, 0))
    qspec1 = pl.BlockSpec((_TB, qs), lambda t, nt=nt: (t + nt, 0))
    return pl.pallas_call(
        _final_kernel,
        grid=(nt,),
        in_specs=[qspec0] * 4 + [qspec1] * 4 + [
            pl.BlockSpec((_TB, 4), lambda t: (t, 0)),
            pl.BlockSpec((1, OUT), lambda t: (0, 0)),
            pl.BlockSpec((1, OUT), lambda t: (0, 0)),
            pl.BlockSpec((1, OUT), lambda t: (0, 0)),
        ],
        out_specs=pl.BlockSpec((_TB, OUT), lambda t: (t, 0)),
        out_shape=jax.ShapeDtypeStruct((T, OUT), _F32),
    )(*ygq, *ygq, pp, b_out.reshape(1, OUT), g_out.reshape(1, OUT),
      be_out.reshape(1, OUT))


def kernel(x, W_map, b_map, W_router, b_router, W1, b1, g1, be1, W2, b2, g2,
           be2, W3, b3, g3, be3, W_out, b_out, g_out, be_out):
    x2d = x.reshape(T, D)
    xb = x2d.astype(_BF16)

    scores = _router(x2d, xb, W_map, b_map, W_router, b_router)

    r = jax.lax.broadcasted_iota(_I32, (T, T), 0)
    c = jax.lax.broadcasted_iota(_I32, (T, T), 1)
    tri = jnp.where(c < r, 1.0, 0.0).astype(_BF16)

    posr, pp, emap_w, nblk11, ent = _routing(scores, tri)
    emap = emap_w.reshape(128)
    nblk = nblk11.reshape(1)

    yq = _ffn(emap, nblk, xb, posr, W1, b1, g1, be1, W2, b2, g2, be2,
              W3, b3, g3, be3, W_out)
    idx = posr.reshape(1, 2 * T).astype(_I32)
    ygq = _sc_gather(yq, idx)
    outs = _final(ygq, pp, b_out, g_out, be_out)
    return outs.reshape(B, T, OUT), ent[0, 0]
